# baseline (speedup 1.0000x reference)
import functools

import jax
import jax.numpy as jnp
from jax import lax
from jax.experimental import pallas as pl
from jax.experimental.pallas import tpu as pltpu
from jax.experimental.pallas import tpu_sc as plsc

N = 10000
E = 320000
D = 128
NC = 2
NS = 16
B = 128

NPAD = 10240
EPAD = 327680
EW = EPAD // (NC * NS)
NCH = EW // B
RING = 2
NCH_DEG = EPAD // NS // B
ROWS_PER_TILE = NPAD // NS

_SC_KERNELS = None


def _get_sc_kernels():
    global _SC_KERNELS
    if _SC_KERNELS is None:
        mesh = plsc.VectorSubcoreMesh(
            core_axis_name="c",
            subcore_axis_name="s",
            num_cores=NC,
            num_subcores=NS,
        )
        _SC_KERNELS = (_make_degree_kernel(mesh), _make_message_kernel(mesh))
    return _SC_KERNELS


def _make_degree_kernel(mesh):
    return functools.partial(
        pl.kernel,
        out_type=jax.ShapeDtypeStruct((NC, NPAD), jnp.float32),
        mesh=mesh,
        scratch_types=[
            [pltpu.VMEM((B,), jnp.int32)] * 2,
            pltpu.VMEM((B,), jnp.float32),
            pltpu.VMEM((ROWS_PER_TILE,), jnp.float32),
            pltpu.VMEM_SHARED((NPAD,), jnp.float32),
            [pltpu.SemaphoreType.DMA] * 2,
            [pltpu.SemaphoreType.DMA] * 2,
        ],
    )(_degree_body)


def _degree_body(edges_hbm, deg_hbm, idx, ones_v, zrow_v, deg_acc, lsems, ssems):
    c = lax.axis_index("c")
    s = lax.axis_index("s")
    one = jnp.ones((16,), jnp.float32)
    zero = jnp.zeros((16,), jnp.float32)
    for k in range(B // 16):
        ones_v[pl.ds(k * 16, 16)] = one
    for k in range(ROWS_PER_TILE // 16):
        zrow_v[pl.ds(k * 16, 16)] = zero
    pltpu.sync_copy(zrow_v, deg_acc.at[pl.ds(s * ROWS_PER_TILE, ROWS_PER_TILE)])
    plsc.subcore_barrier()

    base = (s * NC + c) * NCH * B
    for b in range(2):
        pltpu.async_copy(
            edges_hbm.at[0, pl.ds(base + b * B, B)], idx[b], lsems[b]
        )

    def pair(jj, _):
        j0 = 2 * jj
        for b in range(2):
            j = j0 + b
            pltpu.make_async_copy(
                edges_hbm.at[0, pl.ds(base, B)], idx[b], lsems[b]
            ).wait()
            pltpu.async_copy(ones_v, deg_acc.at[idx[b]], ssems[b], add=True)

            @pl.when(j + 2 < NCH)
            def _next():
                pltpu.make_async_copy(
                    ones_v, deg_acc.at[idx[b]], ssems[b]
                ).wait()
                pltpu.async_copy(
                    edges_hbm.at[0, pl.ds(base + (j + 2) * B, B)],
                    idx[b],
                    lsems[b],
                )

        return _

    lax.fori_loop(0, NCH // 2, pair, None)
    for b in range(2):
        pltpu.make_async_copy(ones_v, deg_acc.at[idx[b]], ssems[b]).wait()
    plsc.subcore_barrier()
    pltpu.sync_copy(
        deg_acc.at[pl.ds(s * ROWS_PER_TILE, ROWS_PER_TILE)],
        deg_hbm.at[c, pl.ds(s * ROWS_PER_TILE, ROWS_PER_TILE)],
    )


def _matmul_body(deg_ref, x_ref, w_ref, o_ref):
    norm = lax.rsqrt(jnp.maximum(deg_ref[0] + deg_ref[1], 1.0))
    o_ref[...] = jnp.dot(
        x_ref[...] * norm, w_ref[...], preferred_element_type=jnp.float32
    )


def _scaled_matmul(deg_col, x, w):
    return pl.pallas_call(
        _matmul_body,
        grid=(N // 2000,),
        in_specs=[
            pl.BlockSpec((NC, 2000, 1), lambda i: (0, i, 0)),
            pl.BlockSpec((2000, D), lambda i: (i, 0)),
            pl.BlockSpec((D, D), lambda i: (0, 0)),
        ],
        out_specs=pl.BlockSpec((2000, D), lambda i: (i, 0)),
        out_shape=jax.ShapeDtypeStruct((NPAD, D), jnp.float32),
    )(deg_col, x, w)


def _make_message_kernel(mesh):
    return functools.partial(
        pl.kernel,
        out_type=(
            jax.ShapeDtypeStruct((NC, NPAD, D), jnp.float32),
            jax.ShapeDtypeStruct((NC, NPAD), jnp.float32),
        ),
        mesh=mesh,
        scratch_types=[
            pltpu.VMEM((EW,), jnp.int32),
            [pltpu.VMEM((B,), jnp.int32)] * RING,
            [pltpu.VMEM((B, D), jnp.float32)] * RING,
            pltpu.VMEM((16, D), jnp.float32),
            pltpu.VMEM((B,), jnp.float32),
            pltpu.VMEM((ROWS_PER_TILE,), jnp.float32),
            pltpu.VMEM_SHARED((NPAD, D), jnp.float32),
            pltpu.VMEM_SHARED((NPAD,), jnp.float32),
            [pltpu.SemaphoreType.DMA] * RING,
            [pltpu.SemaphoreType.DMA] * RING,
            [pltpu.SemaphoreType.DMA] * RING,
            [pltpu.SemaphoreType.DMA] * RING,
        ],
    )(_message_body)


def _message_body(
    h_hbm, edges_hbm, out_hbm, deg_hbm,
    sidx, didx, rows, zbuf, ones_v, zrow_v, acc, deg_acc,
    gsems, ssems, lsems, dsems,
):
    c = lax.axis_index("c")
    s = lax.axis_index("s")
    wid = s * NC + c
    base = wid * EW

    one = jnp.ones((16,), jnp.float32)
    zero = jnp.zeros((16,), jnp.float32)
    for k in range(B // 16):
        ones_v[pl.ds(k * 16, 16)] = one
    for k in range(ROWS_PER_TILE // 16):
        zrow_v[pl.ds(k * 16, 16)] = zero

    def zrow_body(r, _):
        for k in range(D // 16):
            zbuf[r, pl.ds(k * 16, 16)] = zero
        return _

    lax.fori_loop(0, 16, zrow_body, None)

    def zcopy(k, _):
        pltpu.sync_copy(zbuf, acc.at[pl.ds(s * ROWS_PER_TILE + k * 16, 16)])
        return _

    lax.fori_loop(0, ROWS_PER_TILE // 16, zcopy, None)
    pltpu.sync_copy(zrow_v, deg_acc.at[pl.ds(s * ROWS_PER_TILE, ROWS_PER_TILE)])
    plsc.subcore_barrier()

    pltpu.sync_copy(edges_hbm.at[0, pl.ds(base, EW)], sidx)

    for b in range(RING):
        pltpu.async_copy(
            edges_hbm.at[1, pl.ds(base + b * B, B)], didx[b], lsems[b]
        )
        pltpu.async_copy(
            h_hbm.at[sidx.at[pl.ds(b * B, B)]], rows[b], gsems[b]
        )

    def super_iter(jj, _):
        j0 = jj * RING
        for b in range(RING):
            j = j0 + b
            pltpu.make_async_copy(
                h_hbm.at[sidx.at[pl.ds(0, B)]], rows[b], gsems[b]
            ).wait()
            pltpu.make_async_copy(
                edges_hbm.at[1, pl.ds(base, B)], didx[b], lsems[b]
            ).wait()
            pltpu.async_copy(rows[b], acc.at[didx[b]], ssems[b], add=True)
            pltpu.async_copy(ones_v, deg_acc.at[didx[b]], dsems[b], add=True)

            @pl.when(j + RING < NCH)
            def _next():
                pltpu.make_async_copy(
                    rows[b], acc.at[didx[b]], ssems[b]
                ).wait()
                pltpu.make_async_copy(
                    ones_v, deg_acc.at[didx[b]], dsems[b]
                ).wait()
                pltpu.async_copy(
                    edges_hbm.at[1, pl.ds(base + (j + RING) * B, B)],
                    didx[b],
                    lsems[b],
                )
                pltpu.async_copy(
                    h_hbm.at[sidx.at[pl.ds((j + RING) * B, B)]],
                    rows[b],
                    gsems[b],
                )

        return _

    lax.fori_loop(0, NCH // RING, super_iter, None)
    for b in range(RING):
        pltpu.make_async_copy(rows[b], acc.at[didx[b]], ssems[b]).wait()
        pltpu.make_async_copy(ones_v, deg_acc.at[didx[b]], dsems[b]).wait()
    plsc.subcore_barrier()
    pltpu.sync_copy(
        acc.at[pl.ds(s * ROWS_PER_TILE, ROWS_PER_TILE)],
        out_hbm.at[c, pl.ds(s * ROWS_PER_TILE, ROWS_PER_TILE)],
    )
    pltpu.sync_copy(
        deg_acc.at[pl.ds(s * ROWS_PER_TILE, ROWS_PER_TILE)],
        deg_hbm.at[c, pl.ds(s * ROWS_PER_TILE, ROWS_PER_TILE)],
    )


def _finalize_body(deg_ref, parts_ref, b_ref, o_ref):
    norm = lax.rsqrt(jnp.maximum(deg_ref[0] + deg_ref[1], 1.0))
    o_ref[...] = (parts_ref[0] + parts_ref[1]) * norm + b_ref[...]


def _finalize(deg_col, parts, bias_row):
    return pl.pallas_call(
        _finalize_body,
        grid=(N // 2000,),
        in_specs=[
            pl.BlockSpec((NC, 2000, 1), lambda i: (0, i, 0)),
            pl.BlockSpec((NC, 2000, D), lambda i: (0, i, 0)),
            pl.BlockSpec((1, D), lambda i: (0, 0)),
        ],
        out_specs=pl.BlockSpec((2000, D), lambda i: (i, 0)),
        out_shape=jax.ShapeDtypeStruct((N, D), jnp.float32),
    )(deg_col, parts, bias_row)


def kernel(x, edge_index, W, b):
    pad = N + (jnp.arange(EPAD - E, dtype=jnp.int32) % (NPAD - N))
    edges2 = jnp.concatenate(
        [edge_index.astype(jnp.int32), jnp.broadcast_to(pad, (2, EPAD - E))],
        axis=1,
    )

    degree_kernel, message_kernel = _get_sc_kernels()
    deg_out_parts = degree_kernel(edges2)
    h = _scaled_matmul(deg_out_parts.reshape(NC, NPAD, 1), x, W)
    parts, deg_in_parts = message_kernel(h, edges2)
    return _finalize(
        deg_in_parts.reshape(NC, NPAD, 1), parts, b.reshape(1, D)
    )
